# Initial kernel scaffold; baseline (speedup 1.0000x reference)
import jax
import jax.numpy as jnp
from jax.experimental import pallas as pl


def kernel(x, bank_x, bank_y, seed_time, bank_seed_time, Wq, bq, Wk, bk, msg_W1, msg_b1, msg_W2, msg_b2, upd_W1, upd_b1, upd_W2, upd_b2):
    raise NotImplementedError("write your pallas kernel here")


# speedup vs baseline: 4.1471x; 4.1471x over previous
import functools

import jax
import jax.numpy as jnp
from jax import lax
from jax.experimental import pallas as pl
from jax.experimental.pallas import tpu as pltpu
from jax.experimental.pallas import tpu_sc as plsc

NQ = 4096
NB = 16384
C = 128
KK = 20
NEG_INF = float("-inf")

_RB = 256
_CB = 2048


def _proj_bank_body(rows_ref, w_ref, bq_ref, y_ref, w1c_ref, q_ref, g_ref):
    o = lax.dot_general(rows_ref[...], w_ref[...], (((1,), (1,)), ((), ())),
                        preferred_element_type=jnp.float32)
    q_ref[...] = o[:, :C] + bq_ref[...]
    g_ref[...] = o[:, C:] + y_ref[...] * w1c_ref[...]


def _proj_feat_body(rows_ref, w_ref, bk_ref, b1_ref, k_ref, a_ref):
    o = lax.dot_general(rows_ref[...], w_ref[...], (((1,), (1,)), ((), ())),
                        preferred_element_type=jnp.float32)
    k_ref[...] = o[:, :C] + bk_ref[...]
    a_ref[...] = o[:, C:] + b1_ref[...]


def _sim_body(k_ref, q_ref, st_ref, bst_ref, out_ref):
    s = lax.dot_general(k_ref[...], q_ref[...], (((1,), (1,)), ((), ())),
                        preferred_element_type=jnp.float32)
    mask = st_ref[...] < bst_ref[...]
    out_ref[...] = jnp.where(mask, s, NEG_INF)


def _topk_body(s_ref, sc_ref, idx_ref, ssum_ref, scr_ref):
    scr_ref[...] = s_ref[...]
    lane = lax.broadcasted_iota(jnp.int32, (_RB, KK), 1)

    def step(t, carry):
        vals, idxs = carry
        s = scr_ref[...]
        m = jnp.max(s, axis=1, keepdims=True)
        iota = lax.broadcasted_iota(jnp.int32, s.shape, 1)
        cand = jnp.where(s == m, iota, jnp.int32(2 ** 30))
        pos = jnp.min(cand, axis=1, keepdims=True)
        scr_ref[...] = jnp.where(iota == pos, NEG_INF, s)
        vals = jnp.where(lane == t, m, vals)
        idxs = jnp.where(lane == t, pos, idxs)
        return vals, idxs

    vals0 = jnp.full((_RB, KK), NEG_INF, jnp.float32)
    idxs0 = jnp.zeros((_RB, KK), jnp.int32)
    vals, idxs = lax.fori_loop(0, KK, step, (vals0, idxs0))

    m0 = jnp.max(vals, axis=1, keepdims=True)
    finite = m0 > NEG_INF
    e = jnp.where(vals > NEG_INF,
                  jnp.exp(vals - jnp.where(finite, m0, 0.0)), 0.0)
    z = jnp.sum(e, axis=1, keepdims=True)
    sc = jnp.where(z > 0, e / jnp.where(z > 0, z, 1.0), 0.0)
    sc_ref[...] = sc
    idx_ref[...] = idxs
    ssum_ref[...] = jnp.sum(sc, axis=1, keepdims=True)


def _gather_g_rows(g_tbl, idx_flat):
    info = plsc.get_sparse_core_info()
    nw = info.num_cores * info.num_subcores
    n = idx_flat.shape[0]
    b_per_w = n // nw
    chunk = 256
    nchunk = b_per_w // chunk
    mesh = plsc.VectorSubcoreMesh(core_axis_name="c", subcore_axis_name="s")

    @functools.partial(
        pl.kernel, mesh=mesh,
        out_type=jax.ShapeDtypeStruct((n, C), jnp.float32),
        scratch_types=[
            pltpu.VMEM((chunk,), jnp.int32),
            pltpu.VMEM((chunk, C), jnp.float32),
            pltpu.SemaphoreType.DMA,
        ],
    )
    def k(g_hbm, idx_hbm, out_hbm, idx_v, rows_v, sem):
        wid = lax.axis_index("s") * info.num_cores + lax.axis_index("c")
        base = wid * b_per_w
        for ci in range(nchunk):
            off = base + ci * chunk
            pltpu.sync_copy(idx_hbm.at[pl.ds(off, chunk)], idx_v)
            pltpu.async_copy(g_hbm.at[idx_v], rows_v, sem).wait()
            pltpu.sync_copy(rows_v, out_hbm.at[pl.ds(off, chunk)])

    return k(g_tbl, idx_flat)


def _final_body(x_ref, g2_ref, a_ref, sc_ref, s_ref,
                w2_ref, b2_ref, uw1_ref, ub1_ref, uw2_ref, ub2_ref, out_ref):
    i = pl.program_id(0)

    @pl.when(i < NQ // _RB)
    def _():
        g = g2_ref[...].reshape(_RB, KK, C)
        pre = a_ref[...][:, None, :] + g
        rel = jnp.maximum(pre, 0.0)
        r = jnp.sum(sc_ref[...][:, :, None] * rel, axis=1)
        h = lax.dot_general(r, w2_ref[...], (((1,), (1,)), ((), ())),
                            preferred_element_type=jnp.float32)
        h = h + s_ref[...] * b2_ref[...]
        u = jnp.concatenate([x_ref[...], h], axis=1)
        t1 = lax.dot_general(u, uw1_ref[...], (((1,), (1,)), ((), ())),
                             preferred_element_type=jnp.float32) + ub1_ref[...]
        t1 = jnp.maximum(t1, 0.0)
        upd = lax.dot_general(t1, uw2_ref[...], (((1,), (1,)), ((), ())),
                              preferred_element_type=jnp.float32) + ub2_ref[...]
        out_ref[...] = x_ref[...] + upd

    @pl.when(i >= NQ // _RB)
    def _():
        out_ref[...] = x_ref[...]


def kernel(x, bank_x, bank_y, seed_time, bank_seed_time,
           Wq, bq, Wk, bk,
           msg_W1, msg_b1, msg_W2, msg_b2,
           upd_W1, upd_b1, upd_W2, upd_b2):
    feature = x[:NQ]

    w_bank = jnp.concatenate([Wq, msg_W1[:, C:2 * C]], axis=0)
    w_feat = jnp.concatenate([Wk, msg_W1[:, :C]], axis=0)
    w1c = msg_W1[:, 2 * C].reshape(1, C)
    yb = bank_y.reshape(NB, 1)

    pb = 1024
    q_tbl, g_tbl = pl.pallas_call(
        _proj_bank_body,
        grid=(NB // pb,),
        in_specs=[
            pl.BlockSpec((pb, C), lambda i: (i, 0)),
            pl.BlockSpec((2 * C, C), lambda i: (0, 0)),
            pl.BlockSpec((1, C), lambda i: (0, 0)),
            pl.BlockSpec((pb, 1), lambda i: (i, 0)),
            pl.BlockSpec((1, C), lambda i: (0, 0)),
        ],
        out_specs=[pl.BlockSpec((pb, C), lambda i: (i, 0)),
                   pl.BlockSpec((pb, C), lambda i: (i, 0))],
        out_shape=[jax.ShapeDtypeStruct((NB, C), jnp.float32),
                   jax.ShapeDtypeStruct((NB, C), jnp.float32)],
    )(bank_x, w_bank, bq.reshape(1, C), yb, w1c)

    k_tbl, a_tbl = pl.pallas_call(
        _proj_feat_body,
        grid=(NQ // pb,),
        in_specs=[
            pl.BlockSpec((pb, C), lambda i: (i, 0)),
            pl.BlockSpec((2 * C, C), lambda i: (0, 0)),
            pl.BlockSpec((1, C), lambda i: (0, 0)),
            pl.BlockSpec((1, C), lambda i: (0, 0)),
        ],
        out_specs=[pl.BlockSpec((pb, C), lambda i: (i, 0)),
                   pl.BlockSpec((pb, C), lambda i: (i, 0))],
        out_shape=[jax.ShapeDtypeStruct((NQ, C), jnp.float32),
                   jax.ShapeDtypeStruct((NQ, C), jnp.float32)],
    )(feature, w_feat, bk.reshape(1, C), msg_b1.reshape(1, C))

    stf = seed_time.astype(jnp.float32).reshape(NQ, 1)
    bstf = bank_seed_time.astype(jnp.float32).reshape(1, NB)
    sim = pl.pallas_call(
        _sim_body,
        grid=(NQ // _RB, NB // _CB),
        in_specs=[
            pl.BlockSpec((_RB, C), lambda i, j: (i, 0)),
            pl.BlockSpec((_CB, C), lambda i, j: (j, 0)),
            pl.BlockSpec((_RB, 1), lambda i, j: (i, 0)),
            pl.BlockSpec((1, _CB), lambda i, j: (0, j)),
        ],
        out_specs=pl.BlockSpec((_RB, _CB), lambda i, j: (i, j)),
        out_shape=jax.ShapeDtypeStruct((NQ, NB), jnp.float32),
    )(k_tbl, q_tbl, stf, bstf)

    sc, idx, ssum = pl.pallas_call(
        _topk_body,
        grid=(NQ // _RB,),
        in_specs=[pl.BlockSpec((_RB, NB), lambda i: (i, 0))],
        out_specs=[pl.BlockSpec((_RB, KK), lambda i: (i, 0)),
                   pl.BlockSpec((_RB, KK), lambda i: (i, 0)),
                   pl.BlockSpec((_RB, 1), lambda i: (i, 0))],
        out_shape=[jax.ShapeDtypeStruct((NQ, KK), jnp.float32),
                   jax.ShapeDtypeStruct((NQ, KK), jnp.int32),
                   jax.ShapeDtypeStruct((NQ, 1), jnp.float32)],
        scratch_shapes=[pltpu.VMEM((_RB, NB), jnp.float32)],
    )(sim)

    g2 = _gather_g_rows(g_tbl, idx.reshape(-1))

    nrb = NQ // _RB
    out = pl.pallas_call(
        _final_body,
        grid=(x.shape[0] // _RB,),
        in_specs=[
            pl.BlockSpec((_RB, C), lambda i: (i, 0)),
            pl.BlockSpec((_RB * KK, C), lambda i: (jnp.minimum(i, nrb - 1), 0)),
            pl.BlockSpec((_RB, C), lambda i: (jnp.minimum(i, nrb - 1), 0)),
            pl.BlockSpec((_RB, KK), lambda i: (jnp.minimum(i, nrb - 1), 0)),
            pl.BlockSpec((_RB, 1), lambda i: (jnp.minimum(i, nrb - 1), 0)),
            pl.BlockSpec((C, C), lambda i: (0, 0)),
            pl.BlockSpec((1, C), lambda i: (0, 0)),
            pl.BlockSpec((C, 2 * C), lambda i: (0, 0)),
            pl.BlockSpec((1, C), lambda i: (0, 0)),
            pl.BlockSpec((C, C), lambda i: (0, 0)),
            pl.BlockSpec((1, C), lambda i: (0, 0)),
        ],
        out_specs=pl.BlockSpec((_RB, C), lambda i: (i, 0)),
        out_shape=jax.ShapeDtypeStruct(x.shape, jnp.float32),
    )(x, g2, a_tbl, sc, ssum,
      msg_W2, msg_b2.reshape(1, C), upd_W1, upd_b1.reshape(1, C),
      upd_W2, upd_b2.reshape(1, C))

    return out


# speedup vs baseline: 7.6627x; 1.8477x over previous
import functools

import jax
import jax.numpy as jnp
from jax import lax
from jax.experimental import pallas as pl
from jax.experimental.pallas import tpu as pltpu
from jax.experimental.pallas import tpu_sc as plsc

NQ = 4096
NB = 16384
C = 128
KK = 20
CH = 32
NCH = NB // CH
RE = 144
NEG_INF = float("-inf")

_RB = 256
_CB = 4096


def _proj_bank_body(rows_ref, w_ref, bq_ref, y_ref, w1c_ref, q_ref, g_ref):
    o = lax.dot_general(rows_ref[...], w_ref[...], (((1,), (1,)), ((), ())),
                        preferred_element_type=jnp.float32)
    q_ref[...] = o[:, :C] + bq_ref[...]
    g_ref[...] = o[:, C:] + y_ref[...] * w1c_ref[...]


def _proj_feat_body(rows_ref, w_ref, bk_ref, b1_ref, k_ref, a_ref):
    o = lax.dot_general(rows_ref[...], w_ref[...], (((1,), (1,)), ((), ())),
                        preferred_element_type=jnp.float32)
    k_ref[...] = o[:, :C] + bk_ref[...]
    a_ref[...] = o[:, C:] + b1_ref[...]


def _sim_body(k_ref, q_ref, st_ref, bst_ref, out_ref, cm_ref):
    s = lax.dot_general(k_ref[...], q_ref[...], (((1,), (1,)), ((), ())),
                        preferred_element_type=jnp.float32)
    mask = st_ref[...] < bst_ref[...]
    sm = jnp.where(mask, s, NEG_INF)
    out_ref[...] = sm.reshape(_RB, _CB // C, C)
    j = pl.program_id(1)
    nc = _CB // CH
    cm_ref[:, pl.ds(j * nc, nc)] = jnp.max(sm.reshape(_RB, nc, CH), axis=2)


def _rev(v):
    return jnp.flip(v, 0)


def _merge2(ka, va, kb, vb):
    krb, vrb = _rev(kb), _rev(vb)
    m = ka >= krb
    khi = jnp.where(m, ka, krb)
    vhi = jnp.where(m, va, vrb)
    klo = jnp.where(m, krb, ka)
    vlo = jnp.where(m, vrb, va)
    khi, vhi = plsc.sort_key_val(khi, vhi, descending=True)
    klo, vlo = plsc.sort_key_val(klo, vlo, descending=True)
    return khi, vhi, klo, vlo


def _mergetop(p, q):
    kp0, vp0, kp1, vp1 = p
    kq0, vq0, kq1, vq1 = q
    kr0, vr0 = _rev(kq1), _rev(vq1)
    kr1, vr1 = _rev(kq0), _rev(vq0)
    m0 = kp0 >= kr0
    m1 = kp1 >= kr1
    h0 = jnp.where(m0, kp0, kr0)
    w0 = jnp.where(m0, vp0, vr0)
    h1 = jnp.where(m1, kp1, kr1)
    w1 = jnp.where(m1, vp1, vr1)
    mm = h0 >= h1
    x = jnp.where(mm, h0, h1)
    xv = jnp.where(mm, w0, w1)
    y = jnp.where(mm, h1, h0)
    yv = jnp.where(mm, w1, w0)
    x, xv = plsc.sort_key_val(x, xv, descending=True)
    y, yv = plsc.sort_key_val(y, yv, descending=True)
    return x, xv, y, yv


def _tournament(pairs):
    pools = [_merge2(*pairs[i], *pairs[i + 1]) for i in range(0, len(pairs), 2)]
    while len(pools) > 1:
        pools = [_mergetop(pools[i], pools[i + 1])
                 for i in range(0, len(pools), 2)]
    return pools[0]


def _select_aggregate(cm, s3, g_tbl, a_tbl):
    info = plsc.get_sparse_core_info()
    nw = info.num_cores * info.num_subcores
    rows_w = NQ // nw
    ngrp = NB // C
    mesh = plsc.VectorSubcoreMesh(core_axis_name="c", subcore_axis_name="s")

    @functools.partial(
        pl.kernel, mesh=mesh,
        out_type=[jax.ShapeDtypeStruct((NQ, C), jnp.float32),
                  jax.ShapeDtypeStruct((NQ,), jnp.float32)],
        compiler_params=pltpu.CompilerParams(needs_layout_passes=False),
        scratch_types=[
            pltpu.VMEM((rows_w, NCH), jnp.float32),
            pltpu.VMEM((rows_w, C), jnp.float32),
            pltpu.VMEM((rows_w, C), jnp.float32),
            pltpu.VMEM((rows_w,), jnp.float32),
            pltpu.VMEM((32,), jnp.int32),
            pltpu.VMEM((48,), jnp.int32),
            pltpu.VMEM((32, C), jnp.float32),
            pltpu.VMEM((32,), jnp.int32),
            pltpu.VMEM((48,), jnp.float32),
            pltpu.VMEM((32, C), jnp.float32),
            pltpu.SemaphoreType.DMA,
        ],
    )
    def k(cm_hbm, s3_hbm, g_hbm, a_hbm, r_hbm, s_hbm,
          cmbig, abig, rbuf, sbuf, cidx, crel, crows, gidx, scbuf, grows, sem):
        lanes = lax.iota(jnp.int32, 16)
        wid = lax.axis_index("s") * info.num_cores + lax.axis_index("c")
        base = wid * rows_w
        pltpu.sync_copy(cm_hbm.at[pl.ds(base, rows_w)], cmbig)
        pltpu.sync_copy(a_hbm.at[pl.ds(base, rows_w)], abig)

        def row_body(lr, svacc):
            row = base + lr
            pairs = []
            for v in range(NCH // 16):
                kv = cmbig[lr, pl.ds(v * 16, 16)]
                iv = lanes + (v * 16)
                pairs.append(plsc.sort_key_val(kv, iv, descending=True))
            ck0, cv0, ck1, cv1 = _tournament(pairs)
            cidx[pl.ds(0, 16)] = (cv0 >> 2) + row * ngrp
            cidx[pl.ds(16, 16)] = (cv1 >> 2) + row * ngrp
            crel[pl.ds(16, 16)] = cv0
            crel[pl.ds(32, 16)] = cv1
            pltpu.async_copy(s3_hbm.at[cidx], crows, sem).wait()

            pairs2 = []
            for j in range(32):
                jv = jnp.full((16,), j, jnp.int32)
                cj = plsc.load_gather(crel, [jv + 16])
                off = (cj & 3) * CH
                for h in range(CH // 16):
                    kv = plsc.load_gather(crows, [jv, off + (h * 16) + lanes])
                    iv = cj * CH + (h * 16) + lanes
                    pairs2.append(plsc.sort_key_val(kv, iv, descending=True))
            tk0, ti0, tk1, ti1 = _tournament(pairs2)
            tk1 = jnp.where(lanes < (KK - 16), tk1, NEG_INF)

            m0 = jnp.maximum(jnp.max(tk0, axis=0), jnp.max(tk1, axis=0))
            fin = m0 > NEG_INF
            msafe = jnp.where(fin, m0, 0.0)
            e0 = jnp.where(tk0 > NEG_INF, jnp.exp(tk0 - msafe), 0.0)
            e1 = jnp.where(tk1 > NEG_INF, jnp.exp(tk1 - msafe), 0.0)
            z = jnp.sum(e0, axis=0) + jnp.sum(e1, axis=0)
            zv = jnp.zeros((16,), jnp.float32) + z
            invz = jnp.where(zv > 0.0, 1.0 / jnp.where(zv > 0.0, zv, 1.0), 0.0)
            sc0 = e0 * invz
            sc1 = e1 * invz
            srow = jnp.sum(sc0, axis=0) + jnp.sum(sc1, axis=0)
            scbuf[pl.ds(16, 16)] = sc0
            scbuf[pl.ds(32, 16)] = sc1
            gidx[pl.ds(0, 16)] = ti0
            gidx[pl.ds(16, 16)] = ti1
            pltpu.async_copy(g_hbm.at[gidx], grows, sem).wait()

            arow = [abig[lr, pl.ds(h * 16, 16)] for h in range(C // 16)]
            acc = [jnp.zeros((16,), jnp.float32) for _ in range(C // 16)]
            for j in range(32):
                wj = plsc.load_gather(scbuf, [jnp.full((16,), j + 16, jnp.int32)])
                for h in range(C // 16):
                    gv = grows[j, pl.ds(h * 16, 16)]
                    acc[h] = acc[h] + wj * jnp.maximum(arow[h] + gv, 0.0)
            for h in range(C // 16):
                rbuf[lr, pl.ds(h * 16, 16)] = acc[h]
            svacc = jnp.where(lanes == (lr & 15), srow, svacc)
            sbuf[pl.ds((lr >> 4) * 16, 16)] = svacc
            return svacc

        lax.fori_loop(0, rows_w, row_body, jnp.zeros((16,), jnp.float32))
        pltpu.sync_copy(rbuf, r_hbm.at[pl.ds(base, rows_w)])
        pltpu.sync_copy(sbuf, s_hbm.at[pl.ds(base, rows_w)])

    return k(cm, s3, g_tbl, a_tbl)


def _final_body(x_ref, r_ref, s_ref, w2_ref, b2_ref, uw1_ref, ub1_ref,
                uw2_ref, ub2_ref, out_ref):
    i = pl.program_id(0)

    @pl.when(i < NQ // _RB)
    def _():
        r = r_ref[...]
        s = s_ref[...]
        h = lax.dot_general(r, w2_ref[...], (((1,), (1,)), ((), ())),
                            preferred_element_type=jnp.float32)
        h = h + s * b2_ref[...]
        u = jnp.concatenate([x_ref[...], h], axis=1)
        t1 = lax.dot_general(u, uw1_ref[...], (((1,), (1,)), ((), ())),
                             preferred_element_type=jnp.float32) + ub1_ref[...]
        t1 = jnp.maximum(t1, 0.0)
        upd = lax.dot_general(t1, uw2_ref[...], (((1,), (1,)), ((), ())),
                              preferred_element_type=jnp.float32) + ub2_ref[...]
        out_ref[...] = x_ref[...] + upd

    @pl.when(i >= NQ // _RB)
    def _():
        out_ref[...] = x_ref[...]


def kernel(x, bank_x, bank_y, seed_time, bank_seed_time,
           Wq, bq, Wk, bk,
           msg_W1, msg_b1, msg_W2, msg_b2,
           upd_W1, upd_b1, upd_W2, upd_b2):
    feature = x[:NQ]

    w_bank = jnp.concatenate([Wq, msg_W1[:, C:2 * C]], axis=0)
    w_feat = jnp.concatenate([Wk, msg_W1[:, :C]], axis=0)
    w1c = msg_W1[:, 2 * C].reshape(1, C)
    yb = bank_y.reshape(NB, 1)

    pb = 1024
    q_tbl, g_tbl = pl.pallas_call(
        _proj_bank_body,
        grid=(NB // pb,),
        in_specs=[
            pl.BlockSpec((pb, C), lambda i: (i, 0)),
            pl.BlockSpec((2 * C, C), lambda i: (0, 0)),
            pl.BlockSpec((1, C), lambda i: (0, 0)),
            pl.BlockSpec((pb, 1), lambda i: (i, 0)),
            pl.BlockSpec((1, C), lambda i: (0, 0)),
        ],
        out_specs=[pl.BlockSpec((pb, C), lambda i: (i, 0)),
                   pl.BlockSpec((pb, C), lambda i: (i, 0))],
        out_shape=[jax.ShapeDtypeStruct((NB, C), jnp.float32),
                   jax.ShapeDtypeStruct((NB, C), jnp.float32)],
    )(bank_x, w_bank, bq.reshape(1, C), yb, w1c)

    k_tbl, a_tbl = pl.pallas_call(
        _proj_feat_body,
        grid=(NQ // pb,),
        in_specs=[
            pl.BlockSpec((pb, C), lambda i: (i, 0)),
            pl.BlockSpec((2 * C, C), lambda i: (0, 0)),
            pl.BlockSpec((1, C), lambda i: (0, 0)),
            pl.BlockSpec((1, C), lambda i: (0, 0)),
        ],
        out_specs=[pl.BlockSpec((pb, C), lambda i: (i, 0)),
                   pl.BlockSpec((pb, C), lambda i: (i, 0))],
        out_shape=[jax.ShapeDtypeStruct((NQ, C), jnp.float32),
                   jax.ShapeDtypeStruct((NQ, C), jnp.float32)],
    )(feature, w_feat, bk.reshape(1, C), msg_b1.reshape(1, C))

    stf = seed_time.astype(jnp.float32).reshape(NQ, 1)
    bstf = bank_seed_time.astype(jnp.float32).reshape(1, NB)
    sim, cm = pl.pallas_call(
        _sim_body,
        grid=(NQ // _RB, NB // _CB),
        in_specs=[
            pl.BlockSpec((_RB, C), lambda i, j: (i, 0)),
            pl.BlockSpec((_CB, C), lambda i, j: (j, 0)),
            pl.BlockSpec((_RB, 1), lambda i, j: (i, 0)),
            pl.BlockSpec((1, _CB), lambda i, j: (0, j)),
        ],
        out_specs=[pl.BlockSpec((_RB, _CB // C, C), lambda i, j: (i, j, 0)),
                   pl.BlockSpec((_RB, NCH), lambda i, j: (i, 0))],
        out_shape=[jax.ShapeDtypeStruct((NQ, NB // C, C), jnp.float32),
                   jax.ShapeDtypeStruct((NQ, NCH), jnp.float32)],
    )(k_tbl, q_tbl, stf, bstf)

    s3 = sim.reshape(NQ * (NB // C), C)
    r_agg, svec = _select_aggregate(cm, s3, g_tbl, a_tbl)

    nrb = NQ // _RB
    out = pl.pallas_call(
        _final_body,
        grid=(x.shape[0] // _RB,),
        in_specs=[
            pl.BlockSpec((_RB, C), lambda i: (i, 0)),
            pl.BlockSpec((_RB, C), lambda i: (jnp.minimum(i, nrb - 1), 0)),
            pl.BlockSpec((_RB, 1), lambda i: (jnp.minimum(i, nrb - 1), 0)),
            pl.BlockSpec((C, C), lambda i: (0, 0)),
            pl.BlockSpec((1, C), lambda i: (0, 0)),
            pl.BlockSpec((C, 2 * C), lambda i: (0, 0)),
            pl.BlockSpec((1, C), lambda i: (0, 0)),
            pl.BlockSpec((C, C), lambda i: (0, 0)),
            pl.BlockSpec((1, C), lambda i: (0, 0)),
        ],
        out_specs=pl.BlockSpec((_RB, C), lambda i: (i, 0)),
        out_shape=jax.ShapeDtypeStruct(x.shape, jnp.float32),
    )(x, r_agg, svec.reshape(NQ, 1),
      msg_W2, msg_b2.reshape(1, C), upd_W1, upd_b1.reshape(1, C),
      upd_W2, upd_b2.reshape(1, C))

    return out


# speedup vs baseline: 12.4836x; 1.6291x over previous
import functools

import jax
import jax.numpy as jnp
from jax import lax
from jax.experimental import pallas as pl
from jax.experimental.pallas import tpu as pltpu
from jax.experimental.pallas import tpu_sc as plsc

NQ = 4096
NB = 16384
C = 128
KK = 20
CH = 32
NCH = NB // CH
RE = 144
NEG_INF = float("-inf")

_RB = 256
_CB = 4096


def _proj_bank_body(rows_ref, w_ref, bq_ref, y_ref, w1c_ref, q_ref, g_ref):
    o = lax.dot_general(rows_ref[...], w_ref[...], (((1,), (1,)), ((), ())),
                        preferred_element_type=jnp.float32)
    q_ref[...] = o[:, :C] + bq_ref[...]
    g_ref[...] = o[:, C:] + y_ref[...] * w1c_ref[...]


def _proj_feat_body(rows_ref, w_ref, bk_ref, b1_ref, k_ref, a_ref):
    o = lax.dot_general(rows_ref[...], w_ref[...], (((1,), (1,)), ((), ())),
                        preferred_element_type=jnp.float32)
    k_ref[...] = o[:, :C] + bk_ref[...]
    a_ref[...] = o[:, C:] + b1_ref[...]


def _sim_body(k_ref, q_ref, st_ref, bst_ref, str_ref, bstc_ref, out_ref, cm_ref):
    s = lax.dot_general(k_ref[...], q_ref[...], (((1,), (1,)), ((), ())),
                        preferred_element_type=jnp.float32)
    sm = jnp.where(st_ref[...] < bst_ref[...], s, NEG_INF)
    out_ref[...] = sm.reshape(_RB, _CB // C, C)
    st_t = lax.dot_general(q_ref[...], k_ref[...], (((1,), (1,)), ((), ())),
                           preferred_element_type=jnp.float32)
    smt = jnp.where(str_ref[...] < bstc_ref[...], st_t, NEG_INF)
    cmt = jnp.max(smt.reshape(_CB // CH, CH, _RB), axis=1)
    cm_ref[...] = cmt.T


def _rev(v):
    return jnp.flip(v, 0)


def _merge2(ka, va, kb, vb):
    krb, vrb = _rev(kb), _rev(vb)
    m = ka >= krb
    khi = jnp.where(m, ka, krb)
    vhi = jnp.where(m, va, vrb)
    klo = jnp.where(m, krb, ka)
    vlo = jnp.where(m, vrb, va)
    khi, vhi = plsc.sort_key_val(khi, vhi, descending=True)
    klo, vlo = plsc.sort_key_val(klo, vlo, descending=True)
    return khi, vhi, klo, vlo


def _mergetop(p, q):
    kp0, vp0, kp1, vp1 = p
    kq0, vq0, kq1, vq1 = q
    kr0, vr0 = _rev(kq1), _rev(vq1)
    kr1, vr1 = _rev(kq0), _rev(vq0)
    m0 = kp0 >= kr0
    m1 = kp1 >= kr1
    h0 = jnp.where(m0, kp0, kr0)
    w0 = jnp.where(m0, vp0, vr0)
    h1 = jnp.where(m1, kp1, kr1)
    w1 = jnp.where(m1, vp1, vr1)
    mm = h0 >= h1
    x = jnp.where(mm, h0, h1)
    xv = jnp.where(mm, w0, w1)
    y = jnp.where(mm, h1, h0)
    yv = jnp.where(mm, w1, w0)
    x, xv = plsc.sort_key_val(x, xv, descending=True)
    y, yv = plsc.sort_key_val(y, yv, descending=True)
    return x, xv, y, yv


def _tournament(pairs):
    pools = [_merge2(*pairs[i], *pairs[i + 1]) for i in range(0, len(pairs), 2)]
    while len(pools) > 1:
        pools = [_mergetop(pools[i], pools[i + 1])
                 for i in range(0, len(pools), 2)]
    return pools[0]


def _select_aggregate(cm, s3, g_tbl, a_tbl):
    info = plsc.get_sparse_core_info()
    nw = info.num_cores * info.num_subcores
    rows_w = NQ // nw
    ngrp = NB // C
    mesh = plsc.VectorSubcoreMesh(core_axis_name="c", subcore_axis_name="s")

    @functools.partial(
        pl.kernel, mesh=mesh,
        out_type=[jax.ShapeDtypeStruct((NQ, C), jnp.float32),
                  jax.ShapeDtypeStruct((NQ,), jnp.float32)],
        compiler_params=pltpu.CompilerParams(needs_layout_passes=False),
        scratch_types=[
            pltpu.VMEM((rows_w, NCH), jnp.float32),
            pltpu.VMEM((rows_w, C), jnp.float32),
            pltpu.VMEM((rows_w, C), jnp.float32),
            pltpu.VMEM((rows_w,), jnp.float32),
            pltpu.VMEM((32,), jnp.int32),
            pltpu.VMEM((48,), jnp.int32),
            pltpu.VMEM((32, C), jnp.float32),
            pltpu.VMEM((32,), jnp.int32),
            pltpu.VMEM((48,), jnp.float32),
            pltpu.VMEM((32, C), jnp.float32),
            pltpu.SemaphoreType.DMA,
        ],
    )
    def k(cm_hbm, s3_hbm, g_hbm, a_hbm, r_hbm, s_hbm,
          cmbig, abig, rbuf, sbuf, cidx, crel, crows, gidx, scbuf, grows, sem):
        lanes = lax.iota(jnp.int32, 16)
        wid = lax.axis_index("s") * info.num_cores + lax.axis_index("c")
        base = wid * rows_w
        pltpu.sync_copy(cm_hbm.at[pl.ds(base, rows_w)], cmbig)
        pltpu.sync_copy(a_hbm.at[pl.ds(base, rows_w)], abig)

        def row_body(lr, svacc):
            row = base + lr
            pairs = []
            for v in range(NCH // 16):
                kv = cmbig[lr, pl.ds(v * 16, 16)]
                iv = lanes + (v * 16)
                pairs.append(plsc.sort_key_val(kv, iv, descending=True))
            ck0, cv0, ck1, cv1 = _tournament(pairs)
            cidx[pl.ds(0, 16)] = (cv0 >> 2) + row * ngrp
            cidx[pl.ds(16, 16)] = (cv1 >> 2) + row * ngrp
            crel[pl.ds(16, 16)] = cv0
            crel[pl.ds(32, 16)] = cv1
            pltpu.async_copy(s3_hbm.at[cidx], crows, sem).wait()

            pairs2 = []
            for j in range(32):
                jv = jnp.full((16,), j, jnp.int32)
                cj = plsc.load_gather(crel, [jv + 16])
                off = (cj & 3) * CH
                for h in range(CH // 16):
                    kv = plsc.load_gather(crows, [jv, off + (h * 16) + lanes])
                    iv = cj * CH + (h * 16) + lanes
                    pairs2.append(plsc.sort_key_val(kv, iv, descending=True))
            tk0, ti0, tk1, ti1 = _tournament(pairs2)
            tk1 = jnp.where(lanes < (KK - 16), tk1, NEG_INF)

            m0 = jnp.maximum(jnp.max(tk0, axis=0), jnp.max(tk1, axis=0))
            fin = m0 > NEG_INF
            msafe = jnp.where(fin, m0, 0.0)
            e0 = jnp.where(tk0 > NEG_INF, jnp.exp(tk0 - msafe), 0.0)
            e1 = jnp.where(tk1 > NEG_INF, jnp.exp(tk1 - msafe), 0.0)
            z = jnp.sum(e0, axis=0) + jnp.sum(e1, axis=0)
            zv = jnp.zeros((16,), jnp.float32) + z
            invz = jnp.where(zv > 0.0, 1.0 / jnp.where(zv > 0.0, zv, 1.0), 0.0)
            sc0 = e0 * invz
            sc1 = e1 * invz
            srow = jnp.sum(sc0, axis=0) + jnp.sum(sc1, axis=0)
            scbuf[pl.ds(16, 16)] = sc0
            scbuf[pl.ds(32, 16)] = sc1
            gidx[pl.ds(0, 16)] = ti0
            gidx[pl.ds(16, 16)] = ti1
            pltpu.async_copy(g_hbm.at[gidx], grows, sem).wait()

            arow = [abig[lr, pl.ds(h * 16, 16)] for h in range(C // 16)]
            acc = [jnp.zeros((16,), jnp.float32) for _ in range(C // 16)]
            for j in range(32):
                wj = plsc.load_gather(scbuf, [jnp.full((16,), j + 16, jnp.int32)])
                for h in range(C // 16):
                    gv = grows[j, pl.ds(h * 16, 16)]
                    acc[h] = acc[h] + wj * jnp.maximum(arow[h] + gv, 0.0)
            for h in range(C // 16):
                rbuf[lr, pl.ds(h * 16, 16)] = acc[h]
            svacc = jnp.where(lanes == (lr & 15), srow, svacc)
            sbuf[pl.ds((lr >> 4) * 16, 16)] = svacc
            return svacc

        lax.fori_loop(0, rows_w, row_body, jnp.zeros((16,), jnp.float32))
        pltpu.sync_copy(rbuf, r_hbm.at[pl.ds(base, rows_w)])
        pltpu.sync_copy(sbuf, s_hbm.at[pl.ds(base, rows_w)])

    return k(cm, s3, g_tbl, a_tbl)


def _final_body(x_ref, r_ref, s_ref, w2_ref, b2_ref, uw1_ref, ub1_ref,
                uw2_ref, ub2_ref, out_ref):
    i = pl.program_id(0)

    @pl.when(i < NQ // _RB)
    def _():
        r = r_ref[...]
        s = s_ref[...]
        h = lax.dot_general(r, w2_ref[...], (((1,), (1,)), ((), ())),
                            preferred_element_type=jnp.float32)
        h = h + s * b2_ref[...]
        u = jnp.concatenate([x_ref[...], h], axis=1)
        t1 = lax.dot_general(u, uw1_ref[...], (((1,), (1,)), ((), ())),
                             preferred_element_type=jnp.float32) + ub1_ref[...]
        t1 = jnp.maximum(t1, 0.0)
        upd = lax.dot_general(t1, uw2_ref[...], (((1,), (1,)), ((), ())),
                              preferred_element_type=jnp.float32) + ub2_ref[...]
        out_ref[...] = x_ref[...] + upd

    @pl.when(i >= NQ // _RB)
    def _():
        out_ref[...] = x_ref[...]


def kernel(x, bank_x, bank_y, seed_time, bank_seed_time,
           Wq, bq, Wk, bk,
           msg_W1, msg_b1, msg_W2, msg_b2,
           upd_W1, upd_b1, upd_W2, upd_b2):
    feature = x[:NQ]

    w_bank = jnp.concatenate([Wq, msg_W1[:, C:2 * C]], axis=0)
    w_feat = jnp.concatenate([Wk, msg_W1[:, :C]], axis=0)
    w1c = msg_W1[:, 2 * C].reshape(1, C)
    yb = bank_y.reshape(NB, 1)

    pb = 1024
    q_tbl, g_tbl = pl.pallas_call(
        _proj_bank_body,
        grid=(NB // pb,),
        in_specs=[
            pl.BlockSpec((pb, C), lambda i: (i, 0)),
            pl.BlockSpec((2 * C, C), lambda i: (0, 0)),
            pl.BlockSpec((1, C), lambda i: (0, 0)),
            pl.BlockSpec((pb, 1), lambda i: (i, 0)),
            pl.BlockSpec((1, C), lambda i: (0, 0)),
        ],
        out_specs=[pl.BlockSpec((pb, C), lambda i: (i, 0)),
                   pl.BlockSpec((pb, C), lambda i: (i, 0))],
        out_shape=[jax.ShapeDtypeStruct((NB, C), jnp.float32),
                   jax.ShapeDtypeStruct((NB, C), jnp.float32)],
    )(bank_x, w_bank, bq.reshape(1, C), yb, w1c)

    k_tbl, a_tbl = pl.pallas_call(
        _proj_feat_body,
        grid=(NQ // pb,),
        in_specs=[
            pl.BlockSpec((pb, C), lambda i: (i, 0)),
            pl.BlockSpec((2 * C, C), lambda i: (0, 0)),
            pl.BlockSpec((1, C), lambda i: (0, 0)),
            pl.BlockSpec((1, C), lambda i: (0, 0)),
        ],
        out_specs=[pl.BlockSpec((pb, C), lambda i: (i, 0)),
                   pl.BlockSpec((pb, C), lambda i: (i, 0))],
        out_shape=[jax.ShapeDtypeStruct((NQ, C), jnp.float32),
                   jax.ShapeDtypeStruct((NQ, C), jnp.float32)],
    )(feature, w_feat, bk.reshape(1, C), msg_b1.reshape(1, C))

    stf = seed_time.astype(jnp.float32).reshape(NQ, 1)
    bstf = bank_seed_time.astype(jnp.float32).reshape(1, NB)
    str_ = seed_time.astype(jnp.float32).reshape(1, NQ)
    bstc = bank_seed_time.astype(jnp.float32).reshape(NB, 1)
    sim, cm = pl.pallas_call(
        _sim_body,
        grid=(NQ // _RB, NB // _CB),
        in_specs=[
            pl.BlockSpec((_RB, C), lambda i, j: (i, 0)),
            pl.BlockSpec((_CB, C), lambda i, j: (j, 0)),
            pl.BlockSpec((_RB, 1), lambda i, j: (i, 0)),
            pl.BlockSpec((1, _CB), lambda i, j: (0, j)),
            pl.BlockSpec((1, _RB), lambda i, j: (0, i)),
            pl.BlockSpec((_CB, 1), lambda i, j: (j, 0)),
        ],
        out_specs=[pl.BlockSpec((_RB, _CB // C, C), lambda i, j: (i, j, 0)),
                   pl.BlockSpec((_RB, _CB // CH), lambda i, j: (i, j))],
        out_shape=[jax.ShapeDtypeStruct((NQ, NB // C, C), jnp.float32),
                   jax.ShapeDtypeStruct((NQ, NCH), jnp.float32)],
    )(k_tbl, q_tbl, stf, bstf, str_, bstc)

    s3 = sim.reshape(NQ * (NB // C), C)
    r_agg, svec = _select_aggregate(cm, s3, g_tbl, a_tbl)

    nrb = NQ // _RB
    out = pl.pallas_call(
        _final_body,
        grid=(x.shape[0] // _RB,),
        in_specs=[
            pl.BlockSpec((_RB, C), lambda i: (i, 0)),
            pl.BlockSpec((_RB, C), lambda i: (jnp.minimum(i, nrb - 1), 0)),
            pl.BlockSpec((_RB, 1), lambda i: (jnp.minimum(i, nrb - 1), 0)),
            pl.BlockSpec((C, C), lambda i: (0, 0)),
            pl.BlockSpec((1, C), lambda i: (0, 0)),
            pl.BlockSpec((C, 2 * C), lambda i: (0, 0)),
            pl.BlockSpec((1, C), lambda i: (0, 0)),
            pl.BlockSpec((C, C), lambda i: (0, 0)),
            pl.BlockSpec((1, C), lambda i: (0, 0)),
        ],
        out_specs=pl.BlockSpec((_RB, C), lambda i: (i, 0)),
        out_shape=jax.ShapeDtypeStruct(x.shape, jnp.float32),
    )(x, r_agg, svec.reshape(NQ, 1),
      msg_W2, msg_b2.reshape(1, C), upd_W1, upd_b1.reshape(1, C),
      upd_W2, upd_b2.reshape(1, C))

    return out


# speedup vs baseline: 14.2529x; 1.1417x over previous
import functools

import jax
import jax.numpy as jnp
from jax import lax
from jax.experimental import pallas as pl
from jax.experimental.pallas import tpu as pltpu
from jax.experimental.pallas import tpu_sc as plsc

NQ = 4096
NB = 16384
C = 128
KK = 20
CH = 32
NCH = NB // CH
RE = 144
NEG_INF = float("-inf")

_RB = 256
_CB = 4096


def _proj_bank_body(rows_ref, w_ref, bq_ref, y_ref, w1c_ref, q_ref, g_ref):
    o = lax.dot_general(rows_ref[...], w_ref[...], (((1,), (1,)), ((), ())),
                        preferred_element_type=jnp.float32)
    q_ref[...] = o[:, :C] + bq_ref[...]
    g_ref[...] = o[:, C:] + y_ref[...] * w1c_ref[...]


def _proj_feat_body(rows_ref, w_ref, bk_ref, b1_ref, k_ref, a_ref):
    o = lax.dot_general(rows_ref[...], w_ref[...], (((1,), (1,)), ((), ())),
                        preferred_element_type=jnp.float32)
    k_ref[...] = o[:, :C] + bk_ref[...]
    a_ref[...] = o[:, C:] + b1_ref[...]


def _sim_body(k_ref, q_ref, st_ref, bst_ref, str_ref, bstc_ref, out_ref, cm_ref):
    s = lax.dot_general(k_ref[...], q_ref[...], (((1,), (1,)), ((), ())),
                        preferred_element_type=jnp.float32)
    sm = jnp.where(st_ref[...] < bst_ref[...], s, NEG_INF)
    out_ref[...] = sm.reshape(_RB, _CB // C, C)
    st_t = lax.dot_general(q_ref[...], k_ref[...], (((1,), (1,)), ((), ())),
                           preferred_element_type=jnp.float32)
    smt = jnp.where(str_ref[...] < bstc_ref[...], st_t, NEG_INF)
    cmt = jnp.max(smt.reshape(_CB // CH, CH, _RB), axis=1)
    cm_ref[...] = cmt.T


def _rev(v):
    return jnp.flip(v, 0)


def _merge2(ka, va, kb, vb):
    krb, vrb = _rev(kb), _rev(vb)
    m = ka >= krb
    khi = jnp.where(m, ka, krb)
    vhi = jnp.where(m, va, vrb)
    klo = jnp.where(m, krb, ka)
    vlo = jnp.where(m, vrb, va)
    khi, vhi = plsc.sort_key_val(khi, vhi, descending=True)
    klo, vlo = plsc.sort_key_val(klo, vlo, descending=True)
    return khi, vhi, klo, vlo


def _mergetop(p, q):
    kp0, vp0, kp1, vp1 = p
    kq0, vq0, kq1, vq1 = q
    kr0, vr0 = _rev(kq1), _rev(vq1)
    kr1, vr1 = _rev(kq0), _rev(vq0)
    m0 = kp0 >= kr0
    m1 = kp1 >= kr1
    h0 = jnp.where(m0, kp0, kr0)
    w0 = jnp.where(m0, vp0, vr0)
    h1 = jnp.where(m1, kp1, kr1)
    w1 = jnp.where(m1, vp1, vr1)
    mm = h0 >= h1
    x = jnp.where(mm, h0, h1)
    xv = jnp.where(mm, w0, w1)
    y = jnp.where(mm, h1, h0)
    yv = jnp.where(mm, w1, w0)
    x, xv = plsc.sort_key_val(x, xv, descending=True)
    y, yv = plsc.sort_key_val(y, yv, descending=True)
    return x, xv, y, yv


def _tournament(pairs):
    pools = [_merge2(*pairs[i], *pairs[i + 1]) for i in range(0, len(pairs), 2)]
    while len(pools) > 1:
        nxt = [_mergetop(pools[i], pools[i + 1])
               for i in range(0, len(pools) - 1, 2)]
        if len(pools) % 2:
            nxt.append(pools[-1])
        pools = nxt
    return pools[0]


def _select_aggregate(cm, s3, g_tbl, a_tbl):
    info = plsc.get_sparse_core_info()
    nw = info.num_cores * info.num_subcores
    rows_w = NQ // nw
    ngrp = NB // C
    mesh = plsc.VectorSubcoreMesh(core_axis_name="c", subcore_axis_name="s")

    @functools.partial(
        pl.kernel, mesh=mesh,
        out_type=[jax.ShapeDtypeStruct((NQ, C), jnp.float32),
                  jax.ShapeDtypeStruct((NQ,), jnp.float32)],
        compiler_params=pltpu.CompilerParams(needs_layout_passes=False),
        scratch_types=[
            pltpu.VMEM((rows_w, NCH), jnp.float32),
            pltpu.VMEM((rows_w, C), jnp.float32),
            pltpu.VMEM((rows_w, C), jnp.float32),
            pltpu.VMEM((rows_w,), jnp.float32),
            pltpu.VMEM((32,), jnp.int32),
            pltpu.VMEM((48,), jnp.int32),
            pltpu.VMEM((KK, C), jnp.float32),
            pltpu.VMEM((32,), jnp.int32),
            pltpu.VMEM((48,), jnp.float32),
            pltpu.VMEM((KK, C), jnp.float32),
            pltpu.SemaphoreType.DMA,
        ],
    )
    def k(cm_hbm, s3_hbm, g_hbm, a_hbm, r_hbm, s_hbm,
          cmbig, abig, rbuf, sbuf, cidx, crel, crows, gidx, scbuf, grows, sem):
        lanes = lax.iota(jnp.int32, 16)
        wid = lax.axis_index("s") * info.num_cores + lax.axis_index("c")
        base = wid * rows_w
        pltpu.sync_copy(cm_hbm.at[pl.ds(base, rows_w)], cmbig)
        pltpu.sync_copy(a_hbm.at[pl.ds(base, rows_w)], abig)

        def row_body(lr, svacc):
            row = base + lr
            pairs = []
            for v in range(NCH // 16):
                kv = cmbig[lr, pl.ds(v * 16, 16)]
                iv = lanes + (v * 16)
                pairs.append(plsc.sort_key_val(kv, iv, descending=True))
            ck0, cv0, ck1, cv1 = _tournament(pairs)
            cidx[pl.ds(0, 16)] = (cv0 >> 2) + row * ngrp
            cidx[pl.ds(16, 16)] = (cv1 >> 2) + row * ngrp
            crel[pl.ds(16, 16)] = cv0
            crel[pl.ds(32, 16)] = cv1
            pltpu.async_copy(s3_hbm.at[cidx.at[pl.ds(0, KK)]], crows, sem).wait()

            pairs2 = []
            for j in range(KK):
                jv = jnp.full((16,), j, jnp.int32)
                cj = plsc.load_gather(crel, [jv + 16])
                off = (cj & 3) * CH
                for h in range(CH // 16):
                    kv = plsc.load_gather(crows, [jv, off + (h * 16) + lanes])
                    iv = cj * CH + (h * 16) + lanes
                    pairs2.append(plsc.sort_key_val(kv, iv, descending=True))
            tk0, ti0, tk1, ti1 = _tournament(pairs2)
            tk1 = jnp.where(lanes < (KK - 16), tk1, NEG_INF)

            m0 = jnp.maximum(jnp.max(tk0, axis=0), jnp.max(tk1, axis=0))
            fin = m0 > NEG_INF
            msafe = jnp.where(fin, m0, 0.0)
            e0 = jnp.where(tk0 > NEG_INF, jnp.exp(tk0 - msafe), 0.0)
            e1 = jnp.where(tk1 > NEG_INF, jnp.exp(tk1 - msafe), 0.0)
            z = jnp.sum(e0, axis=0) + jnp.sum(e1, axis=0)
            zv = jnp.zeros((16,), jnp.float32) + z
            invz = jnp.where(zv > 0.0, 1.0 / jnp.where(zv > 0.0, zv, 1.0), 0.0)
            sc0 = e0 * invz
            sc1 = e1 * invz
            srow = jnp.sum(sc0, axis=0) + jnp.sum(sc1, axis=0)
            scbuf[pl.ds(16, 16)] = sc0
            scbuf[pl.ds(32, 16)] = sc1
            gidx[pl.ds(0, 16)] = ti0
            gidx[pl.ds(16, 16)] = ti1
            pltpu.async_copy(g_hbm.at[gidx.at[pl.ds(0, KK)]], grows, sem).wait()

            arow = [abig[lr, pl.ds(h * 16, 16)] for h in range(C // 16)]
            acc = [jnp.zeros((16,), jnp.float32) for _ in range(C // 16)]
            for j in range(KK):
                wj = plsc.load_gather(scbuf, [jnp.full((16,), j + 16, jnp.int32)])
                for h in range(C // 16):
                    gv = grows[j, pl.ds(h * 16, 16)]
                    acc[h] = acc[h] + wj * jnp.maximum(arow[h] + gv, 0.0)
            for h in range(C // 16):
                rbuf[lr, pl.ds(h * 16, 16)] = acc[h]
            svacc = jnp.where(lanes == (lr & 15), srow, svacc)
            sbuf[pl.ds((lr >> 4) * 16, 16)] = svacc
            return svacc

        lax.fori_loop(0, rows_w, row_body, jnp.zeros((16,), jnp.float32))
        pltpu.sync_copy(rbuf, r_hbm.at[pl.ds(base, rows_w)])
        pltpu.sync_copy(sbuf, s_hbm.at[pl.ds(base, rows_w)])

    return k(cm, s3, g_tbl, a_tbl)


def _final_body(x_ref, r_ref, s_ref, w2_ref, b2_ref, uw1_ref, ub1_ref,
                uw2_ref, ub2_ref, out_ref):
    i = pl.program_id(0)

    @pl.when(i < NQ // _RB)
    def _():
        r = r_ref[...]
        s = s_ref[...]
        h = lax.dot_general(r, w2_ref[...], (((1,), (1,)), ((), ())),
                            preferred_element_type=jnp.float32)
        h = h + s * b2_ref[...]
        u = jnp.concatenate([x_ref[...], h], axis=1)
        t1 = lax.dot_general(u, uw1_ref[...], (((1,), (1,)), ((), ())),
                             preferred_element_type=jnp.float32) + ub1_ref[...]
        t1 = jnp.maximum(t1, 0.0)
        upd = lax.dot_general(t1, uw2_ref[...], (((1,), (1,)), ((), ())),
                              preferred_element_type=jnp.float32) + ub2_ref[...]
        out_ref[...] = x_ref[...] + upd

    @pl.when(i >= NQ // _RB)
    def _():
        out_ref[...] = x_ref[...]


def kernel(x, bank_x, bank_y, seed_time, bank_seed_time,
           Wq, bq, Wk, bk,
           msg_W1, msg_b1, msg_W2, msg_b2,
           upd_W1, upd_b1, upd_W2, upd_b2):
    feature = x[:NQ]

    w_bank = jnp.concatenate([Wq, msg_W1[:, C:2 * C]], axis=0)
    w_feat = jnp.concatenate([Wk, msg_W1[:, :C]], axis=0)
    w1c = msg_W1[:, 2 * C].reshape(1, C)
    yb = bank_y.reshape(NB, 1)

    pb = 1024
    q_tbl, g_tbl = pl.pallas_call(
        _proj_bank_body,
        grid=(NB // pb,),
        in_specs=[
            pl.BlockSpec((pb, C), lambda i: (i, 0)),
            pl.BlockSpec((2 * C, C), lambda i: (0, 0)),
            pl.BlockSpec((1, C), lambda i: (0, 0)),
            pl.BlockSpec((pb, 1), lambda i: (i, 0)),
            pl.BlockSpec((1, C), lambda i: (0, 0)),
        ],
        out_specs=[pl.BlockSpec((pb, C), lambda i: (i, 0)),
                   pl.BlockSpec((pb, C), lambda i: (i, 0))],
        out_shape=[jax.ShapeDtypeStruct((NB, C), jnp.float32),
                   jax.ShapeDtypeStruct((NB, C), jnp.float32)],
    )(bank_x, w_bank, bq.reshape(1, C), yb, w1c)

    k_tbl, a_tbl = pl.pallas_call(
        _proj_feat_body,
        grid=(NQ // pb,),
        in_specs=[
            pl.BlockSpec((pb, C), lambda i: (i, 0)),
            pl.BlockSpec((2 * C, C), lambda i: (0, 0)),
            pl.BlockSpec((1, C), lambda i: (0, 0)),
            pl.BlockSpec((1, C), lambda i: (0, 0)),
        ],
        out_specs=[pl.BlockSpec((pb, C), lambda i: (i, 0)),
                   pl.BlockSpec((pb, C), lambda i: (i, 0))],
        out_shape=[jax.ShapeDtypeStruct((NQ, C), jnp.float32),
                   jax.ShapeDtypeStruct((NQ, C), jnp.float32)],
    )(feature, w_feat, bk.reshape(1, C), msg_b1.reshape(1, C))

    stf = seed_time.astype(jnp.float32).reshape(NQ, 1)
    bstf = bank_seed_time.astype(jnp.float32).reshape(1, NB)
    str_ = seed_time.astype(jnp.float32).reshape(1, NQ)
    bstc = bank_seed_time.astype(jnp.float32).reshape(NB, 1)
    sim, cm = pl.pallas_call(
        _sim_body,
        grid=(NQ // _RB, NB // _CB),
        in_specs=[
            pl.BlockSpec((_RB, C), lambda i, j: (i, 0)),
            pl.BlockSpec((_CB, C), lambda i, j: (j, 0)),
            pl.BlockSpec((_RB, 1), lambda i, j: (i, 0)),
            pl.BlockSpec((1, _CB), lambda i, j: (0, j)),
            pl.BlockSpec((1, _RB), lambda i, j: (0, i)),
            pl.BlockSpec((_CB, 1), lambda i, j: (j, 0)),
        ],
        out_specs=[pl.BlockSpec((_RB, _CB // C, C), lambda i, j: (i, j, 0)),
                   pl.BlockSpec((_RB, _CB // CH), lambda i, j: (i, j))],
        out_shape=[jax.ShapeDtypeStruct((NQ, NB // C, C), jnp.float32),
                   jax.ShapeDtypeStruct((NQ, NCH), jnp.float32)],
    )(k_tbl, q_tbl, stf, bstf, str_, bstc)

    s3 = sim.reshape(NQ * (NB // C), C)
    r_agg, svec = _select_aggregate(cm, s3, g_tbl, a_tbl)

    nrb = NQ // _RB
    out = pl.pallas_call(
        _final_body,
        grid=(x.shape[0] // _RB,),
        in_specs=[
            pl.BlockSpec((_RB, C), lambda i: (i, 0)),
            pl.BlockSpec((_RB, C), lambda i: (jnp.minimum(i, nrb - 1), 0)),
            pl.BlockSpec((_RB, 1), lambda i: (jnp.minimum(i, nrb - 1), 0)),
            pl.BlockSpec((C, C), lambda i: (0, 0)),
            pl.BlockSpec((1, C), lambda i: (0, 0)),
            pl.BlockSpec((C, 2 * C), lambda i: (0, 0)),
            pl.BlockSpec((1, C), lambda i: (0, 0)),
            pl.BlockSpec((C, C), lambda i: (0, 0)),
            pl.BlockSpec((1, C), lambda i: (0, 0)),
        ],
        out_specs=pl.BlockSpec((_RB, C), lambda i: (i, 0)),
        out_shape=jax.ShapeDtypeStruct(x.shape, jnp.float32),
    )(x, r_agg, svec.reshape(NQ, 1),
      msg_W2, msg_b2.reshape(1, C), upd_W1, upd_b1.reshape(1, C),
      upd_W2, upd_b2.reshape(1, C))

    return out


# speedup vs baseline: 19.8877x; 1.3953x over previous
import functools

import jax
import jax.numpy as jnp
from jax import lax
from jax.experimental import pallas as pl
from jax.experimental.pallas import tpu as pltpu
from jax.experimental.pallas import tpu_sc as plsc

NQ = 4096
NB = 16384
C = 128
KK = 20
KP = 24
CH = 32
NCH = NB // CH
RE = 144
NEG_INF = float("-inf")

_RB = 256
_CB = 4096


def _proj_bank_body(rows_ref, w_ref, bq_ref, y_ref, w1c_ref, q_ref, g_ref):
    o = lax.dot_general(rows_ref[...], w_ref[...], (((1,), (1,)), ((), ())),
                        preferred_element_type=jnp.float32)
    q_ref[...] = o[:, :C] + bq_ref[...]
    g_ref[...] = o[:, C:] + y_ref[...] * w1c_ref[...]


def _proj_feat_body(rows_ref, w_ref, bk_ref, b1_ref, k_ref, a_ref):
    o = lax.dot_general(rows_ref[...], w_ref[...], (((1,), (1,)), ((), ())),
                        preferred_element_type=jnp.float32)
    k_ref[...] = o[:, :C] + bk_ref[...]
    a_ref[...] = o[:, C:] + b1_ref[...]


def _sim_body(k_ref, q_ref, st_ref, bst_ref, str_ref, bstc_ref, out_ref, cm_ref):
    s = lax.dot_general(k_ref[...], q_ref[...], (((1,), (1,)), ((), ())),
                        preferred_element_type=jnp.float32)
    sm = jnp.where(st_ref[...] < bst_ref[...], s, NEG_INF)
    out_ref[...] = sm.reshape(_RB, _CB // C, C)
    st_t = lax.dot_general(q_ref[...], k_ref[...], (((1,), (1,)), ((), ())),
                           preferred_element_type=jnp.float32)
    smt = jnp.where(str_ref[...] < bstc_ref[...], st_t, NEG_INF)
    cmt = jnp.max(smt.reshape(_CB // CH, CH, _RB), axis=1)
    cm_ref[...] = cmt.T


def _rev(v):
    return jnp.flip(v, 0)


def _merge2(ka, va, kb, vb):
    krb, vrb = _rev(kb), _rev(vb)
    m = ka >= krb
    khi = jnp.where(m, ka, krb)
    vhi = jnp.where(m, va, vrb)
    klo = jnp.where(m, krb, ka)
    vlo = jnp.where(m, vrb, va)
    khi, vhi = plsc.sort_key_val(khi, vhi, descending=True)
    klo, vlo = plsc.sort_key_val(klo, vlo, descending=True)
    return khi, vhi, klo, vlo


def _mergetop(p, q):
    kp0, vp0, kp1, vp1 = p
    kq0, vq0, kq1, vq1 = q
    kr0, vr0 = _rev(kq1), _rev(vq1)
    kr1, vr1 = _rev(kq0), _rev(vq0)
    m0 = kp0 >= kr0
    m1 = kp1 >= kr1
    h0 = jnp.where(m0, kp0, kr0)
    w0 = jnp.where(m0, vp0, vr0)
    h1 = jnp.where(m1, kp1, kr1)
    w1 = jnp.where(m1, vp1, vr1)
    mm = h0 >= h1
    x = jnp.where(mm, h0, h1)
    xv = jnp.where(mm, w0, w1)
    y = jnp.where(mm, h1, h0)
    yv = jnp.where(mm, w1, w0)
    x, xv = plsc.sort_key_val(x, xv, descending=True)
    y, yv = plsc.sort_key_val(y, yv, descending=True)
    return x, xv, y, yv


def _tournament(pairs):
    pools = [_merge2(*pairs[i], *pairs[i + 1]) for i in range(0, len(pairs), 2)]
    while len(pools) > 1:
        nxt = [_mergetop(pools[i], pools[i + 1])
               for i in range(0, len(pools) - 1, 2)]
        if len(pools) % 2:
            nxt.append(pools[-1])
        pools = nxt
    return pools[0]


def _select_aggregate(cm, s3, g_tbl, a_tbl):
    info = plsc.get_sparse_core_info()
    nw = info.num_cores * info.num_subcores
    rows_w = NQ // nw
    ngrp = NB // C
    mesh = plsc.VectorSubcoreMesh(core_axis_name="c", subcore_axis_name="s")

    @functools.partial(
        pl.kernel, mesh=mesh,
        out_type=[jax.ShapeDtypeStruct((NQ, C), jnp.float32),
                  jax.ShapeDtypeStruct((NQ,), jnp.float32)],
        compiler_params=pltpu.CompilerParams(needs_layout_passes=False),
        scratch_types=[
            pltpu.VMEM((rows_w, NCH), jnp.float32),
            pltpu.VMEM((rows_w, C), jnp.float32),
            pltpu.VMEM((rows_w, C), jnp.float32),
            pltpu.VMEM((rows_w,), jnp.float32),
            pltpu.VMEM((64,), jnp.int32),
            pltpu.VMEM((96,), jnp.int32),
            pltpu.VMEM((2 * KP, C), jnp.float32),
            pltpu.VMEM((64,), jnp.int32),
            pltpu.VMEM((96,), jnp.float32),
            pltpu.VMEM((2 * KP, C), jnp.float32),
            pltpu.SemaphoreType.DMA,
            pltpu.SemaphoreType.DMA,
        ],
    )
    def k(cm_hbm, s3_hbm, g_hbm, a_hbm, r_hbm, s_hbm,
          cmbig, abig, rbuf, sbuf, cidx, crel, crows, gidx, scbuf, grows,
          csem, gsem):
        lanes = lax.iota(jnp.int32, 16)
        wid = lax.axis_index("s") * info.num_cores + lax.axis_index("c")
        base = wid * rows_w
        pltpu.sync_copy(cm_hbm.at[pl.ds(base, rows_w)], cmbig)
        pltpu.sync_copy(a_hbm.at[pl.ds(base, rows_w)], abig)

        def row_body(it, svacc):
            lr = jnp.minimum(it, rows_w - 1)
            b1 = it & 1
            row = base + lr
            pairs = []
            for v in range(NCH // 16):
                kv = cmbig[lr, pl.ds(v * 16, 16)]
                iv = lanes + (v * 16)
                pairs.append(plsc.sort_key_val(kv, iv, descending=True))
            ck0, cv0, ck1, cv1 = _tournament(pairs)
            cidx[pl.ds(b1 * 32, 16)] = (cv0 >> 2) + row * ngrp
            cidx[pl.ds(b1 * 32 + 16, 16)] = (cv1 >> 2) + row * ngrp
            crel[pl.ds(b1 * 48 + 16, 16)] = cv0
            crel[pl.ds(b1 * 48 + 32, 16)] = cv1

            @pl.when(it < rows_w)
            def _():
                pltpu.async_copy(
                    s3_hbm.at[cidx.at[pl.ds(b1 * 32, KP)]],
                    crows.at[pl.ds(b1 * KP, KP)], csem)

            r1 = jnp.clip(it - 1, 0, rows_w - 1)
            bp = r1 & 1
            s2_act = jnp.logical_and(it >= 1, it <= rows_w)

            @pl.when(s2_act)
            def _():
                pltpu.make_async_copy(
                    s3_hbm.at[pl.ds(0, KP)],
                    crows.at[pl.ds(bp * KP, KP)], csem).wait()

            pairs2 = []
            for j in range(KK):
                jv = jnp.full((16,), j, jnp.int32)
                cj = plsc.load_gather(crel, [jv + (bp * 48 + 16)])
                off = (cj & 3) * CH
                for h in range(CH // 16):
                    kv = plsc.load_gather(
                        crows, [jv + bp * KP, off + (h * 16) + lanes])
                    iv = cj * CH + (h * 16) + lanes
                    pairs2.append(plsc.sort_key_val(kv, iv, descending=True))
            tk0, ti0, tk1, ti1 = _tournament(pairs2)
            tk1 = jnp.where(lanes < (KK - 16), tk1, NEG_INF)
            m0 = jnp.maximum(jnp.max(tk0, axis=0), jnp.max(tk1, axis=0))
            fin = m0 > NEG_INF
            msafe = jnp.where(fin, m0, 0.0)
            e0 = jnp.where(tk0 > NEG_INF, jnp.exp(tk0 - msafe), 0.0)
            e1 = jnp.where(tk1 > NEG_INF, jnp.exp(tk1 - msafe), 0.0)
            z = jnp.sum(e0, axis=0) + jnp.sum(e1, axis=0)
            zv = jnp.zeros((16,), jnp.float32) + z
            invz = jnp.where(zv > 0.0, 1.0 / jnp.where(zv > 0.0, zv, 1.0), 0.0)
            sc0 = e0 * invz
            sc1 = e1 * invz
            srow = jnp.sum(sc0, axis=0) + jnp.sum(sc1, axis=0)
            svacc = jnp.where(
                jnp.logical_and(s2_act, lanes == (r1 & 15)), srow, svacc)

            @pl.when(s2_act)
            def _():
                scbuf[pl.ds(bp * 48 + 16, 16)] = sc0
                scbuf[pl.ds(bp * 48 + 32, 16)] = sc1
                gidx[pl.ds(bp * 32, 16)] = ti0
                gidx[pl.ds(bp * 32 + 16, 16)] = ti1
                sbuf[pl.ds((r1 >> 4) * 16, 16)] = svacc
                pltpu.async_copy(
                    g_hbm.at[gidx.at[pl.ds(bp * 32, KP)]],
                    grows.at[pl.ds(bp * KP, KP)], gsem)

            r2 = jnp.clip(it - 2, 0, rows_w - 1)
            b2 = r2 & 1

            @pl.when(it >= 2)
            def _():
                pltpu.make_async_copy(
                    g_hbm.at[pl.ds(0, KP)],
                    grows.at[pl.ds(b2 * KP, KP)], gsem).wait()
                arow = [abig[r2, pl.ds(h * 16, 16)] for h in range(C // 16)]
                acc = [jnp.zeros((16,), jnp.float32) for _ in range(C // 16)]
                for j in range(KK):
                    wj = plsc.load_gather(
                        scbuf, [jnp.full((16,), j + 16, jnp.int32) + b2 * 48])
                    for h in range(C // 16):
                        gv = grows[b2 * KP + j, pl.ds(h * 16, 16)]
                        acc[h] = acc[h] + wj * jnp.maximum(arow[h] + gv, 0.0)
                for h in range(C // 16):
                    rbuf[r2, pl.ds(h * 16, 16)] = acc[h]

            return svacc

        lax.fori_loop(0, rows_w + 2, row_body, jnp.zeros((16,), jnp.float32))
        pltpu.sync_copy(rbuf, r_hbm.at[pl.ds(base, rows_w)])
        pltpu.sync_copy(sbuf, s_hbm.at[pl.ds(base, rows_w)])

    return k(cm, s3, g_tbl, a_tbl)


def _final_body(x_ref, r_ref, s_ref, w2_ref, b2_ref, uw1_ref, ub1_ref,
                uw2_ref, ub2_ref, out_ref):
    i = pl.program_id(0)

    @pl.when(i < NQ // _RB)
    def _():
        r = r_ref[...]
        s = s_ref[...]
        h = lax.dot_general(r, w2_ref[...], (((1,), (1,)), ((), ())),
                            preferred_element_type=jnp.float32)
        h = h + s * b2_ref[...]
        u = jnp.concatenate([x_ref[...], h], axis=1)
        t1 = lax.dot_general(u, uw1_ref[...], (((1,), (1,)), ((), ())),
                             preferred_element_type=jnp.float32) + ub1_ref[...]
        t1 = jnp.maximum(t1, 0.0)
        upd = lax.dot_general(t1, uw2_ref[...], (((1,), (1,)), ((), ())),
                              preferred_element_type=jnp.float32) + ub2_ref[...]
        out_ref[...] = x_ref[...] + upd

    @pl.when(i >= NQ // _RB)
    def _():
        out_ref[...] = x_ref[...]


def kernel(x, bank_x, bank_y, seed_time, bank_seed_time,
           Wq, bq, Wk, bk,
           msg_W1, msg_b1, msg_W2, msg_b2,
           upd_W1, upd_b1, upd_W2, upd_b2):
    feature = x[:NQ]

    w_bank = jnp.concatenate([Wq, msg_W1[:, C:2 * C]], axis=0)
    w_feat = jnp.concatenate([Wk, msg_W1[:, :C]], axis=0)
    w1c = msg_W1[:, 2 * C].reshape(1, C)
    yb = bank_y.reshape(NB, 1)

    pb = 1024
    q_tbl, g_tbl = pl.pallas_call(
        _proj_bank_body,
        grid=(NB // pb,),
        in_specs=[
            pl.BlockSpec((pb, C), lambda i: (i, 0)),
            pl.BlockSpec((2 * C, C), lambda i: (0, 0)),
            pl.BlockSpec((1, C), lambda i: (0, 0)),
            pl.BlockSpec((pb, 1), lambda i: (i, 0)),
            pl.BlockSpec((1, C), lambda i: (0, 0)),
        ],
        out_specs=[pl.BlockSpec((pb, C), lambda i: (i, 0)),
                   pl.BlockSpec((pb, C), lambda i: (i, 0))],
        out_shape=[jax.ShapeDtypeStruct((NB, C), jnp.float32),
                   jax.ShapeDtypeStruct((NB, C), jnp.float32)],
    )(bank_x, w_bank, bq.reshape(1, C), yb, w1c)

    k_tbl, a_tbl = pl.pallas_call(
        _proj_feat_body,
        grid=(NQ // pb,),
        in_specs=[
            pl.BlockSpec((pb, C), lambda i: (i, 0)),
            pl.BlockSpec((2 * C, C), lambda i: (0, 0)),
            pl.BlockSpec((1, C), lambda i: (0, 0)),
            pl.BlockSpec((1, C), lambda i: (0, 0)),
        ],
        out_specs=[pl.BlockSpec((pb, C), lambda i: (i, 0)),
                   pl.BlockSpec((pb, C), lambda i: (i, 0))],
        out_shape=[jax.ShapeDtypeStruct((NQ, C), jnp.float32),
                   jax.ShapeDtypeStruct((NQ, C), jnp.float32)],
    )(feature, w_feat, bk.reshape(1, C), msg_b1.reshape(1, C))

    stf = seed_time.astype(jnp.float32).reshape(NQ, 1)
    bstf = bank_seed_time.astype(jnp.float32).reshape(1, NB)
    str_ = seed_time.astype(jnp.float32).reshape(1, NQ)
    bstc = bank_seed_time.astype(jnp.float32).reshape(NB, 1)
    sim, cm = pl.pallas_call(
        _sim_body,
        grid=(NQ // _RB, NB // _CB),
        in_specs=[
            pl.BlockSpec((_RB, C), lambda i, j: (i, 0)),
            pl.BlockSpec((_CB, C), lambda i, j: (j, 0)),
            pl.BlockSpec((_RB, 1), lambda i, j: (i, 0)),
            pl.BlockSpec((1, _CB), lambda i, j: (0, j)),
            pl.BlockSpec((1, _RB), lambda i, j: (0, i)),
            pl.BlockSpec((_CB, 1), lambda i, j: (j, 0)),
        ],
        out_specs=[pl.BlockSpec((_RB, _CB // C, C), lambda i, j: (i, j, 0)),
                   pl.BlockSpec((_RB, _CB // CH), lambda i, j: (i, j))],
        out_shape=[jax.ShapeDtypeStruct((NQ, NB // C, C), jnp.float32),
                   jax.ShapeDtypeStruct((NQ, NCH), jnp.float32)],
    )(k_tbl, q_tbl, stf, bstf, str_, bstc)

    s3 = sim.reshape(NQ * (NB // C), C)
    r_agg, svec = _select_aggregate(cm, s3, g_tbl, a_tbl)

    nrb = NQ // _RB
    out = pl.pallas_call(
        _final_body,
        grid=(x.shape[0] // _RB,),
        in_specs=[
            pl.BlockSpec((_RB, C), lambda i: (i, 0)),
            pl.BlockSpec((_RB, C), lambda i: (jnp.minimum(i, nrb - 1), 0)),
            pl.BlockSpec((_RB, 1), lambda i: (jnp.minimum(i, nrb - 1), 0)),
            pl.BlockSpec((C, C), lambda i: (0, 0)),
            pl.BlockSpec((1, C), lambda i: (0, 0)),
            pl.BlockSpec((C, 2 * C), lambda i: (0, 0)),
            pl.BlockSpec((1, C), lambda i: (0, 0)),
            pl.BlockSpec((C, C), lambda i: (0, 0)),
            pl.BlockSpec((1, C), lambda i: (0, 0)),
        ],
        out_specs=pl.BlockSpec((_RB, C), lambda i: (i, 0)),
        out_shape=jax.ShapeDtypeStruct(x.shape, jnp.float32),
    )(x, r_agg, svec.reshape(NQ, 1),
      msg_W2, msg_b2.reshape(1, C), upd_W1, upd_b1.reshape(1, C),
      upd_W2, upd_b2.reshape(1, C))

    return out


# speedup vs baseline: 19.9351x; 1.0024x over previous
import functools

import jax
import jax.numpy as jnp
from jax import lax
from jax.experimental import pallas as pl
from jax.experimental.pallas import tpu as pltpu
from jax.experimental.pallas import tpu_sc as plsc

NQ = 4096
NB = 16384
C = 128
KK = 20
KP = 24
CH = 32
NCH = NB // CH
RE = 144
NEG_INF = float("-inf")

_RB = 256
_CB = 4096


def _proj_bank_body(rows_ref, w_ref, bq_ref, y_ref, w1c_ref, q_ref, g_ref):
    o = lax.dot_general(rows_ref[...], w_ref[...], (((1,), (1,)), ((), ())),
                        preferred_element_type=jnp.float32)
    q_ref[...] = o[:, :C] + bq_ref[...]
    g_ref[...] = o[:, C:] + y_ref[...] * w1c_ref[...]


def _proj_feat_body(rows_ref, w_ref, bk_ref, b1_ref, k_ref, a_ref):
    o = lax.dot_general(rows_ref[...], w_ref[...], (((1,), (1,)), ((), ())),
                        preferred_element_type=jnp.float32)
    k_ref[...] = o[:, :C] + bk_ref[...]
    a_ref[...] = o[:, C:] + b1_ref[...]


def _sim_body(k_ref, q_ref, st_ref, bst_ref, str_ref, bstc_ref, out_ref, cm_ref):
    kb = k_ref[...].astype(jnp.bfloat16)
    qb = q_ref[...].astype(jnp.bfloat16)
    s = lax.dot_general(kb, qb, (((1,), (1,)), ((), ())),
                        preferred_element_type=jnp.float32)
    sm = jnp.where(st_ref[...] < bst_ref[...], s, NEG_INF)
    out_ref[...] = sm.reshape(_RB, _CB // C, C)
    st_t = lax.dot_general(qb, kb, (((1,), (1,)), ((), ())),
                           preferred_element_type=jnp.float32)
    smt = jnp.where(str_ref[...] < bstc_ref[...], st_t, NEG_INF)
    cmt = jnp.max(smt.reshape(_CB // CH, CH, _RB), axis=1)
    cm_ref[...] = cmt.T


def _rev(v):
    return jnp.flip(v, 0)


def _merge2(ka, va, kb, vb):
    krb, vrb = _rev(kb), _rev(vb)
    m = ka >= krb
    khi = jnp.where(m, ka, krb)
    vhi = jnp.where(m, va, vrb)
    klo = jnp.where(m, krb, ka)
    vlo = jnp.where(m, vrb, va)
    khi, vhi = plsc.sort_key_val(khi, vhi, descending=True)
    klo, vlo = plsc.sort_key_val(klo, vlo, descending=True)
    return khi, vhi, klo, vlo


def _mergetop(p, q):
    kp0, vp0, kp1, vp1 = p
    kq0, vq0, kq1, vq1 = q
    kr0, vr0 = _rev(kq1), _rev(vq1)
    kr1, vr1 = _rev(kq0), _rev(vq0)
    m0 = kp0 >= kr0
    m1 = kp1 >= kr1
    h0 = jnp.where(m0, kp0, kr0)
    w0 = jnp.where(m0, vp0, vr0)
    h1 = jnp.where(m1, kp1, kr1)
    w1 = jnp.where(m1, vp1, vr1)
    mm = h0 >= h1
    x = jnp.where(mm, h0, h1)
    xv = jnp.where(mm, w0, w1)
    y = jnp.where(mm, h1, h0)
    yv = jnp.where(mm, w1, w0)
    x, xv = plsc.sort_key_val(x, xv, descending=True)
    y, yv = plsc.sort_key_val(y, yv, descending=True)
    return x, xv, y, yv


def _tournament(pairs):
    pools = [_merge2(*pairs[i], *pairs[i + 1]) for i in range(0, len(pairs), 2)]
    while len(pools) > 1:
        nxt = [_mergetop(pools[i], pools[i + 1])
               for i in range(0, len(pools) - 1, 2)]
        if len(pools) % 2:
            nxt.append(pools[-1])
        pools = nxt
    return pools[0]


def _select_aggregate(cm, s3, g_tbl, a_tbl):
    info = plsc.get_sparse_core_info()
    nw = info.num_cores * info.num_subcores
    rows_w = NQ // nw
    ngrp = NB // C
    mesh = plsc.VectorSubcoreMesh(core_axis_name="c", subcore_axis_name="s")

    @functools.partial(
        pl.kernel, mesh=mesh,
        out_type=[jax.ShapeDtypeStruct((NQ, C), jnp.float32),
                  jax.ShapeDtypeStruct((NQ,), jnp.float32)],
        compiler_params=pltpu.CompilerParams(needs_layout_passes=False),
        scratch_types=[
            pltpu.VMEM((rows_w, NCH), jnp.float32),
            pltpu.VMEM((rows_w, C), jnp.float32),
            pltpu.VMEM((rows_w, C), jnp.float32),
            pltpu.VMEM((rows_w,), jnp.float32),
            pltpu.VMEM((64,), jnp.int32),
            pltpu.VMEM((96,), jnp.int32),
            pltpu.VMEM((2 * KP, C), jnp.float32),
            pltpu.VMEM((64,), jnp.int32),
            pltpu.VMEM((96,), jnp.float32),
            pltpu.VMEM((2 * KP, C), jnp.float32),
            pltpu.SemaphoreType.DMA,
            pltpu.SemaphoreType.DMA,
        ],
    )
    def k(cm_hbm, s3_hbm, g_hbm, a_hbm, r_hbm, s_hbm,
          cmbig, abig, rbuf, sbuf, cidx, crel, crows, gidx, scbuf, grows,
          csem, gsem):
        lanes = lax.iota(jnp.int32, 16)
        wid = lax.axis_index("s") * info.num_cores + lax.axis_index("c")
        base = wid * rows_w
        pltpu.sync_copy(cm_hbm.at[pl.ds(base, rows_w)], cmbig)
        pltpu.sync_copy(a_hbm.at[pl.ds(base, rows_w)], abig)

        def row_body(it, svacc):
            lr = jnp.minimum(it, rows_w - 1)
            b1 = it & 1
            row = base + lr
            pairs = []
            for v in range(NCH // 16):
                kv = cmbig[lr, pl.ds(v * 16, 16)]
                iv = lanes + (v * 16)
                pairs.append(plsc.sort_key_val(kv, iv, descending=True))
            ck0, cv0, ck1, cv1 = _tournament(pairs)
            cidx[pl.ds(b1 * 32, 16)] = (cv0 >> 2) + row * ngrp
            cidx[pl.ds(b1 * 32 + 16, 16)] = (cv1 >> 2) + row * ngrp
            crel[pl.ds(b1 * 48 + 16, 16)] = cv0
            crel[pl.ds(b1 * 48 + 32, 16)] = cv1

            @pl.when(it < rows_w)
            def _():
                pltpu.async_copy(
                    s3_hbm.at[cidx.at[pl.ds(b1 * 32, KP)]],
                    crows.at[pl.ds(b1 * KP, KP)], csem)

            r1 = jnp.clip(it - 1, 0, rows_w - 1)
            bp = r1 & 1
            s2_act = jnp.logical_and(it >= 1, it <= rows_w)

            @pl.when(s2_act)
            def _():
                pltpu.make_async_copy(
                    s3_hbm.at[pl.ds(0, KP)],
                    crows.at[pl.ds(bp * KP, KP)], csem).wait()

            pairs2 = []
            for j in range(KK):
                jv = jnp.full((16,), j, jnp.int32)
                cj = plsc.load_gather(crel, [jv + (bp * 48 + 16)])
                off = (cj & 3) * CH
                for h in range(CH // 16):
                    kv = plsc.load_gather(
                        crows, [jv + bp * KP, off + (h * 16) + lanes])
                    iv = cj * CH + (h * 16) + lanes
                    pairs2.append(plsc.sort_key_val(kv, iv, descending=True))
            tk0, ti0, tk1, ti1 = _tournament(pairs2)
            tk1 = jnp.where(lanes < (KK - 16), tk1, NEG_INF)
            m0 = jnp.maximum(jnp.max(tk0, axis=0), jnp.max(tk1, axis=0))
            fin = m0 > NEG_INF
            msafe = jnp.where(fin, m0, 0.0)
            e0 = jnp.where(tk0 > NEG_INF, jnp.exp(tk0 - msafe), 0.0)
            e1 = jnp.where(tk1 > NEG_INF, jnp.exp(tk1 - msafe), 0.0)
            z = jnp.sum(e0, axis=0) + jnp.sum(e1, axis=0)
            zv = jnp.zeros((16,), jnp.float32) + z
            invz = jnp.where(zv > 0.0, 1.0 / jnp.where(zv > 0.0, zv, 1.0), 0.0)
            sc0 = e0 * invz
            sc1 = e1 * invz
            srow = jnp.sum(sc0, axis=0) + jnp.sum(sc1, axis=0)
            svacc = jnp.where(
                jnp.logical_and(s2_act, lanes == (r1 & 15)), srow, svacc)

            @pl.when(s2_act)
            def _():
                scbuf[pl.ds(bp * 48 + 16, 16)] = sc0
                scbuf[pl.ds(bp * 48 + 32, 16)] = sc1
                gidx[pl.ds(bp * 32, 16)] = ti0
                gidx[pl.ds(bp * 32 + 16, 16)] = ti1
                sbuf[pl.ds((r1 >> 4) * 16, 16)] = svacc
                pltpu.async_copy(
                    g_hbm.at[gidx.at[pl.ds(bp * 32, KP)]],
                    grows.at[pl.ds(bp * KP, KP)], gsem)

            r2 = jnp.clip(it - 2, 0, rows_w - 1)
            b2 = r2 & 1

            @pl.when(it >= 2)
            def _():
                pltpu.make_async_copy(
                    g_hbm.at[pl.ds(0, KP)],
                    grows.at[pl.ds(b2 * KP, KP)], gsem).wait()
                arow = [abig[r2, pl.ds(h * 16, 16)] for h in range(C // 16)]
                acc = [jnp.zeros((16,), jnp.float32) for _ in range(C // 16)]
                for j in range(KK):
                    wj = plsc.load_gather(
                        scbuf, [jnp.full((16,), j + 16, jnp.int32) + b2 * 48])
                    for h in range(C // 16):
                        gv = grows[b2 * KP + j, pl.ds(h * 16, 16)]
                        acc[h] = acc[h] + wj * jnp.maximum(arow[h] + gv, 0.0)
                for h in range(C // 16):
                    rbuf[r2, pl.ds(h * 16, 16)] = acc[h]

            return svacc

        lax.fori_loop(0, rows_w + 2, row_body, jnp.zeros((16,), jnp.float32))
        pltpu.sync_copy(rbuf, r_hbm.at[pl.ds(base, rows_w)])
        pltpu.sync_copy(sbuf, s_hbm.at[pl.ds(base, rows_w)])

    return k(cm, s3, g_tbl, a_tbl)


def _final_body(x_ref, r_ref, s_ref, w2_ref, b2_ref, uw1_ref, ub1_ref,
                uw2_ref, ub2_ref, out_ref):
    i = pl.program_id(0)

    @pl.when(i < NQ // _RB)
    def _():
        r = r_ref[...]
        s = s_ref[...]
        h = lax.dot_general(r, w2_ref[...], (((1,), (1,)), ((), ())),
                            preferred_element_type=jnp.float32)
        h = h + s * b2_ref[...]
        u = jnp.concatenate([x_ref[...], h], axis=1)
        t1 = lax.dot_general(u, uw1_ref[...], (((1,), (1,)), ((), ())),
                             preferred_element_type=jnp.float32) + ub1_ref[...]
        t1 = jnp.maximum(t1, 0.0)
        upd = lax.dot_general(t1, uw2_ref[...], (((1,), (1,)), ((), ())),
                              preferred_element_type=jnp.float32) + ub2_ref[...]
        out_ref[...] = x_ref[...] + upd

    @pl.when(i >= NQ // _RB)
    def _():
        out_ref[...] = x_ref[...]


def kernel(x, bank_x, bank_y, seed_time, bank_seed_time,
           Wq, bq, Wk, bk,
           msg_W1, msg_b1, msg_W2, msg_b2,
           upd_W1, upd_b1, upd_W2, upd_b2):
    feature = x[:NQ]

    w_bank = jnp.concatenate([Wq, msg_W1[:, C:2 * C]], axis=0)
    w_feat = jnp.concatenate([Wk, msg_W1[:, :C]], axis=0)
    w1c = msg_W1[:, 2 * C].reshape(1, C)
    yb = bank_y.reshape(NB, 1)

    pb = 1024
    q_tbl, g_tbl = pl.pallas_call(
        _proj_bank_body,
        grid=(NB // pb,),
        in_specs=[
            pl.BlockSpec((pb, C), lambda i: (i, 0)),
            pl.BlockSpec((2 * C, C), lambda i: (0, 0)),
            pl.BlockSpec((1, C), lambda i: (0, 0)),
            pl.BlockSpec((pb, 1), lambda i: (i, 0)),
            pl.BlockSpec((1, C), lambda i: (0, 0)),
        ],
        out_specs=[pl.BlockSpec((pb, C), lambda i: (i, 0)),
                   pl.BlockSpec((pb, C), lambda i: (i, 0))],
        out_shape=[jax.ShapeDtypeStruct((NB, C), jnp.float32),
                   jax.ShapeDtypeStruct((NB, C), jnp.float32)],
    )(bank_x, w_bank, bq.reshape(1, C), yb, w1c)

    k_tbl, a_tbl = pl.pallas_call(
        _proj_feat_body,
        grid=(NQ // pb,),
        in_specs=[
            pl.BlockSpec((pb, C), lambda i: (i, 0)),
            pl.BlockSpec((2 * C, C), lambda i: (0, 0)),
            pl.BlockSpec((1, C), lambda i: (0, 0)),
            pl.BlockSpec((1, C), lambda i: (0, 0)),
        ],
        out_specs=[pl.BlockSpec((pb, C), lambda i: (i, 0)),
                   pl.BlockSpec((pb, C), lambda i: (i, 0))],
        out_shape=[jax.ShapeDtypeStruct((NQ, C), jnp.float32),
                   jax.ShapeDtypeStruct((NQ, C), jnp.float32)],
    )(feature, w_feat, bk.reshape(1, C), msg_b1.reshape(1, C))

    stf = seed_time.astype(jnp.float32).reshape(NQ, 1)
    bstf = bank_seed_time.astype(jnp.float32).reshape(1, NB)
    str_ = seed_time.astype(jnp.float32).reshape(1, NQ)
    bstc = bank_seed_time.astype(jnp.float32).reshape(NB, 1)
    sim, cm = pl.pallas_call(
        _sim_body,
        grid=(NQ // _RB, NB // _CB),
        in_specs=[
            pl.BlockSpec((_RB, C), lambda i, j: (i, 0)),
            pl.BlockSpec((_CB, C), lambda i, j: (j, 0)),
            pl.BlockSpec((_RB, 1), lambda i, j: (i, 0)),
            pl.BlockSpec((1, _CB), lambda i, j: (0, j)),
            pl.BlockSpec((1, _RB), lambda i, j: (0, i)),
            pl.BlockSpec((_CB, 1), lambda i, j: (j, 0)),
        ],
        out_specs=[pl.BlockSpec((_RB, _CB // C, C), lambda i, j: (i, j, 0)),
                   pl.BlockSpec((_RB, _CB // CH), lambda i, j: (i, j))],
        out_shape=[jax.ShapeDtypeStruct((NQ, NB // C, C), jnp.float32),
                   jax.ShapeDtypeStruct((NQ, NCH), jnp.float32)],
    )(k_tbl, q_tbl, stf, bstf, str_, bstc)

    s3 = sim.reshape(NQ * (NB // C), C)
    r_agg, svec = _select_aggregate(cm, s3, g_tbl, a_tbl)

    nrb = NQ // _RB
    out = pl.pallas_call(
        _final_body,
        grid=(x.shape[0] // _RB,),
        in_specs=[
            pl.BlockSpec((_RB, C), lambda i: (i, 0)),
            pl.BlockSpec((_RB, C), lambda i: (jnp.minimum(i, nrb - 1), 0)),
            pl.BlockSpec((_RB, 1), lambda i: (jnp.minimum(i, nrb - 1), 0)),
            pl.BlockSpec((C, C), lambda i: (0, 0)),
            pl.BlockSpec((1, C), lambda i: (0, 0)),
            pl.BlockSpec((C, 2 * C), lambda i: (0, 0)),
            pl.BlockSpec((1, C), lambda i: (0, 0)),
            pl.BlockSpec((C, C), lambda i: (0, 0)),
            pl.BlockSpec((1, C), lambda i: (0, 0)),
        ],
        out_specs=pl.BlockSpec((_RB, C), lambda i: (i, 0)),
        out_shape=jax.ShapeDtypeStruct(x.shape, jnp.float32),
    )(x, r_agg, svec.reshape(NQ, 1),
      msg_W2, msg_b2.reshape(1, C), upd_W1, upd_b1.reshape(1, C),
      upd_W2, upd_b2.reshape(1, C))

    return out
